# Initial kernel scaffold; baseline (speedup 1.0000x reference)
import jax
import jax.numpy as jnp
from jax.experimental import pallas as pl


def kernel(noisy, deno, fflow, bflow):
    raise NotImplementedError("write your pallas kernel here")


# speedup vs baseline: 29.4798x; 29.4798x over previous
import functools

import jax
import jax.numpy as jnp
import numpy as np
from jax.experimental import pallas as pl
from jax.experimental.pallas import tpu as pltpu

B, T, C, H, W = 1, 3, 3, 128, 128
PS, WS, K, STRIDE0 = 7, 11, 10, 2
WC = WS // 2
PAD = PS // 2
NH, NW = H // STRIDE0, W // STRIDE0
EPS = 1e-3
ALPHA = 0.5
NOFF = WS * WS
GSZ = H + 2 * PAD
EXT = WC
ESZ = GSZ + 2 * EXT
NSEL = K - 1
BIG = 1e30


def _box_matrix() -> np.ndarray:
    m = np.zeros((GSZ, NW), np.float32)
    for q in range(NW):
        m[2 * q:2 * q + PS, q] = 1.0
    return m


def _mult_table() -> np.ndarray:
    m1 = np.zeros((NH, WS), np.float32)
    for q in range(NH):
        y = 2 * q
        for dy in range(-WC, WC + 1):
            e = min(max(y + dy, 0), H - 1) - y
            m1[q, e + WC] += 1.0
    mult = np.einsum('ya,xb->abyx', m1, m1).reshape(NOFF, NH, NW)
    mult[NOFF // 2] -= 1.0
    return mult


def _dnls_kernel(noisy_ref, deno_ref, boxT_ref, box_ref, mult_ref, out_ref,
                 srch_scr, dmap_scr, rmap_scr):
    t = pl.program_id(0)

    @pl.when(t == 0)
    def _init():
        out_ref[...] = jnp.zeros((1, 1), jnp.float32)

    srch_scr[...] = ALPHA * noisy_ref[0] + (1.0 - ALPHA) * deno_ref[0]

    boxT = boxT_ref[...]
    box = box_ref[...]

    for off in range(NOFF):
        sy = off // WS
        sx = off % WS
        gs = jnp.zeros((GSZ, GSZ), jnp.float32)
        gr = jnp.zeros((GSZ, GSZ), jnp.float32)
        for c in range(C):
            a_s = srch_scr[c, EXT:EXT + GSZ, EXT:EXT + GSZ]
            b_s = srch_scr[c, sy:sy + GSZ, sx:sx + GSZ]
            d = a_s - b_s
            gs = gs + d * d
            a_r = deno_ref[0, c, EXT:EXT + GSZ, EXT:EXT + GSZ]
            b_r = noisy_ref[0, c, sy:sy + GSZ, sx:sx + GSZ]
            d = a_r - b_r
            gr = gr + d * d
        ds = jnp.dot(jnp.dot(boxT, gs, preferred_element_type=jnp.float32),
                     box, preferred_element_type=jnp.float32)
        dr = jnp.dot(jnp.dot(boxT, gr, preferred_element_type=jnp.float32),
                     box, preferred_element_type=jnp.float32)
        dmap_scr[off] = ds
        rmap_scr[off] = dr

    v = dmap_scr[...]
    sq = jnp.sqrt(jnp.maximum(rmap_scr[...], 0.0) + EPS)
    iota = jax.lax.broadcasted_iota(jnp.int32, (NOFF, NH, NW), 0)

    def select(_, carry):
        acc, rem, mult = carry
        vv = jnp.where(mult > 0.5, v, BIG)
        m = jnp.min(vv, axis=0)
        idx = jnp.min(jnp.where(vv <= m[None], iota, 2 * NOFF), axis=0)
        oneh = iota == idx[None]
        mult_at = jnp.sum(jnp.where(oneh, mult, 0.0), axis=0)
        take = jnp.minimum(mult_at, rem)
        sq_at = jnp.sum(jnp.where(oneh, sq, 0.0), axis=0)
        acc = acc + take * sq_at
        mult = mult - jnp.where(oneh, take[None], 0.0)
        rem = rem - take
        return acc, rem, mult

    acc0 = jnp.zeros((NH, NW), jnp.float32)
    rem0 = jnp.full((NH, NW), float(NSEL), jnp.float32)
    acc, _, _ = jax.lax.fori_loop(0, NSEL, select,
                                  (acc0, rem0, mult_ref[...]))
    out_ref[...] = out_ref[...] + jnp.sum(acc).reshape(1, 1)


@functools.partial(jax.jit, static_argnums=())
def _run(noisy, deno):
    def prep(v):
        vp = jnp.pad(v, ((0, 0), (0, 0), (PAD, PAD), (PAD, PAD)),
                     mode='reflect')
        return jnp.pad(vp, ((0, 0), (0, 0), (EXT, EXT), (EXT, EXT)))

    npe = prep(noisy[0])
    dpe = prep(deno[0])
    box = jnp.asarray(_box_matrix())
    boxT = box.T.copy()
    mult = jnp.asarray(_mult_table())

    total = pl.pallas_call(
        _dnls_kernel,
        grid=(T,),
        in_specs=[
            pl.BlockSpec((1, C, ESZ, ESZ), lambda t: (t, 0, 0, 0)),
            pl.BlockSpec((1, C, ESZ, ESZ), lambda t: (t, 0, 0, 0)),
            pl.BlockSpec((NW, GSZ), lambda t: (0, 0)),
            pl.BlockSpec((GSZ, NW), lambda t: (0, 0)),
            pl.BlockSpec((NOFF, NH, NW), lambda t: (0, 0, 0)),
        ],
        out_specs=pl.BlockSpec((1, 1), lambda t: (0, 0)),
        out_shape=jax.ShapeDtypeStruct((1, 1), jnp.float32),
        scratch_shapes=[
            pltpu.VMEM((C, ESZ, ESZ), jnp.float32),
            pltpu.VMEM((NOFF, NH, NW), jnp.float32),
            pltpu.VMEM((NOFF, NH, NW), jnp.float32),
        ],
    )(npe, dpe, boxT, box, mult)
    return total[0, 0] / float(T * NH * NW * NSEL)


def kernel(noisy, deno, fflow, bflow):
    return _run(noisy, deno)
